# baseline (speedup 1.0000x reference)
import functools

import jax
import jax.numpy as jnp
from jax.experimental import pallas as pl
from jax.experimental.pallas import tpu as pltpu


def _pick_block(n: int, target: int) -> int:
    best = None
    for b in range(8, min(n, target) + 1, 8):
        if n % b == 0:
            best = b
    return best if best is not None else n


_BUF = 3
_X_CHUNKS = 25


def _gcn_body(s, adj_hbm, x_hbm, w_ref, b_ref, out_ref, buf, xb, xc, sems, xsems):
    i = pl.program_id(0)
    nsteps = pl.num_programs(0)
    n_rows_x = xb.shape[0]
    c = n_rows_x // _X_CHUNKS

    def fetch(block, slot):
        return pltpu.make_async_copy(
            adj_hbm.at[pl.ds(block * s, s), :], buf.at[slot], sems.at[slot]
        )

    def xcopy(j):
        return pltpu.make_async_copy(
            x_hbm.at[pl.ds(j * c, c), :], xc.at[j % 2], xsems.at[j % 2]
        )

    @pl.when(i == 0)
    def _prologue():
        for j in range(_BUF - 1):
            fetch(j, j).start()
        xcopy(0).start()
        for j in range(_X_CHUNKS):
            if j + 1 < _X_CHUNKS:
                xcopy(j + 1).start()
            xcopy(j).wait()
            xb[j * c : (j + 1) * c, :] = xc[j % 2].astype(jnp.bfloat16)

    nxt = i + _BUF - 1

    @pl.when(nxt < nsteps)
    def _prefetch():
        fetch(nxt, jax.lax.rem(nxt, _BUF)).start()

    slot = jax.lax.rem(i, _BUF)
    fetch(i, slot).wait()

    g = jnp.dot(buf[slot], xb[...], preferred_element_type=jnp.float32)
    out_ref[...] = (
        jnp.dot(g, w_ref[...], preferred_element_type=jnp.float32)
        + b_ref[...]
    )


@jax.jit
def kernel(x, adj_mat, weight, bias):
    n, d_in = x.shape
    d_out = weight.shape[1]
    s = _pick_block(n, 400)
    chunk = max(n // _X_CHUNKS, 1)
    bias2 = bias.reshape(1, d_out)
    out = pl.pallas_call(
        functools.partial(_gcn_body, s),
        grid=(n // s,),
        in_specs=[
            pl.BlockSpec(memory_space=pl.ANY),
            pl.BlockSpec(memory_space=pl.ANY),
            pl.BlockSpec((d_in, d_out), lambda i: (0, 0)),
            pl.BlockSpec((1, d_out), lambda i: (0, 0)),
        ],
        out_specs=pl.BlockSpec((s, d_out), lambda i: (i, 0)),
        out_shape=jax.ShapeDtypeStruct((n, d_out), jnp.float32),
        scratch_shapes=[
            pltpu.VMEM((_BUF, s, n), jnp.float32),
            pltpu.VMEM((n, d_in), jnp.bfloat16),
            pltpu.VMEM((2, chunk, d_in), jnp.float32),
            pltpu.SemaphoreType.DMA((_BUF,)),
            pltpu.SemaphoreType.DMA((2,)),
        ],
        compiler_params=pltpu.CompilerParams(
            dimension_semantics=("arbitrary",),
            vmem_limit_bytes=128 * 1024 * 1024,
        ),
    )(adj_mat, x, weight, bias2)
    return out


# speedup vs baseline: 1.0345x; 1.0345x over previous
import functools

import jax
import jax.numpy as jnp
from jax.experimental import pallas as pl
from jax.experimental.pallas import tpu as pltpu


def _pick_block(n: int, target: int) -> int:
    best = None
    for b in range(8, min(n, target) + 1, 8):
        if n % b == 0:
            best = b
    return best if best is not None else n


_BUF = 3
_X_CHUNKS = 5


def _gcn_body(s, adj_hbm, x_hbm, w_ref, b_ref, out_ref, buf, xb, xc, sems, xsems):
    i = pl.program_id(0)
    nsteps = pl.num_programs(0)
    n_rows_x = xb.shape[0]
    c = n_rows_x // _X_CHUNKS

    def fetch(block, slot):
        return pltpu.make_async_copy(
            adj_hbm.at[pl.ds(block * s, s), :], buf.at[slot], sems.at[slot]
        )

    @pl.when(i == 0)
    def _prologue():
        for j in range(_BUF - 1):
            fetch(j, j).start()
        for j in range(_X_CHUNKS):
            cp = pltpu.make_async_copy(
                x_hbm.at[pl.ds(j * c, c), :], xc, xsems
            )
            cp.start()
            cp.wait()
            xb[j * c : (j + 1) * c, :] = xc[...].astype(jnp.bfloat16)

    nxt = i + _BUF - 1

    @pl.when(nxt < nsteps)
    def _prefetch():
        fetch(nxt, jax.lax.rem(nxt, _BUF)).start()

    slot = jax.lax.rem(i, _BUF)
    fetch(i, slot).wait()

    g = jnp.dot(buf[slot], xb[...], preferred_element_type=jnp.float32)
    out_ref[...] = (
        jnp.dot(g, w_ref[...], preferred_element_type=jnp.float32)
        + b_ref[...]
    )


@jax.jit
def kernel(x, adj_mat, weight, bias):
    n, d_in = x.shape
    d_out = weight.shape[1]
    s = _pick_block(n, 400)
    chunk = max(n // _X_CHUNKS, 1)
    bias2 = bias.reshape(1, d_out)
    out = pl.pallas_call(
        functools.partial(_gcn_body, s),
        grid=(n // s,),
        in_specs=[
            pl.BlockSpec(memory_space=pl.ANY),
            pl.BlockSpec(memory_space=pl.ANY),
            pl.BlockSpec((d_in, d_out), lambda i: (0, 0)),
            pl.BlockSpec((1, d_out), lambda i: (0, 0)),
        ],
        out_specs=pl.BlockSpec((s, d_out), lambda i: (i, 0)),
        out_shape=jax.ShapeDtypeStruct((n, d_out), jnp.float32),
        scratch_shapes=[
            pltpu.VMEM((_BUF, s, n), jnp.float32),
            pltpu.VMEM((n, d_in), jnp.bfloat16),
            pltpu.VMEM((chunk, d_in), jnp.float32),
            pltpu.SemaphoreType.DMA((_BUF,)),
            pltpu.SemaphoreType.DMA,
        ],
        compiler_params=pltpu.CompilerParams(
            dimension_semantics=("arbitrary",),
            vmem_limit_bytes=128 * 1024 * 1024,
        ),
    )(adj_mat, x, weight, bias2)
    return out
